# baseline (speedup 1.0000x reference)
import jax
import jax.numpy as jnp
from jax import lax
from jax.experimental import pallas as pl
from jax.experimental.pallas import tpu as pltpu
from jax.experimental.pallas import tpu_sc as plsc

_L = 16
_NW = 16


def _make_sc_kernel(B, C, N, KPAD, CH):
    WPB = KPAD // CH
    NV = CH // _L

    def body(flat_h, ind_h, rest_h, out_h,
             ind_v, rest_v, idx0, idx1, val0, val1, part_v, sem):
        w = lax.axis_index("s")
        b = w // WPB
        cin = pltpu.async_copy(ind_h.at[pl.ds(w * CH, CH)], ind_v, sem)
        crest = pltpu.async_copy(rest_h.at[pl.ds(w * 3 * CH, 3 * CH)],
                                 rest_v, sem)
        cin.wait()

        off0 = (b * C + 0) * N
        off1 = (b * C + 1) * N

        def build(j, _):
            n = ind_v[pl.ds(j * _L, _L)].astype(jnp.int32)
            idx0[pl.ds(j * _L, _L)] = n + off0
            idx1[pl.ds(j * _L, _L)] = n + off1
            return 0

        lax.fori_loop(0, NV, build, 0, unroll=False)

        cp0 = pltpu.async_copy(flat_h.at[idx0], val0, sem)
        cp1 = pltpu.async_copy(flat_h.at[idx1], val1, sem)
        crest.wait()
        cp0.wait()
        cp1.wait()

        def accum(j, carry):
            accl, accm = carry
            o = j * _L
            mk = rest_v[pl.ds(o, _L)]
            t0 = rest_v[pl.ds(CH + o, _L)]
            t1 = rest_v[pl.ds(2 * CH + o, _L)]
            accl = accl + (jnp.abs(val0[pl.ds(o, _L)] - t0)
                           + jnp.abs(val1[pl.ds(o, _L)] - t1)) * mk
            accm = accm + mk + mk
            return accl, accm

        zero = jnp.zeros((_L,), jnp.float32)
        accl, accm = lax.fori_loop(0, NV, accum, (zero, zero), unroll=False)

        part_v[pl.ds(0, _L)] = accl
        part_v[pl.ds(_L, _L)] = accm
        pltpu.sync_copy(part_v, out_h.at[pl.ds(w * 2 * _L, 2 * _L)])

    mesh = plsc.VectorSubcoreMesh(
        core_axis_name="c", subcore_axis_name="s", num_cores=1,
        num_subcores=_NW)
    return pl.kernel(
        body,
        out_type=jax.ShapeDtypeStruct((_NW * 2 * _L,), jnp.float32),
        mesh=mesh,
        scratch_types=[
            pltpu.VMEM((CH,), jnp.float32),
            pltpu.VMEM((3 * CH,), jnp.float32),
            pltpu.VMEM((CH,), jnp.int32),
            pltpu.VMEM((CH,), jnp.int32),
            pltpu.VMEM((CH,), jnp.float32),
            pltpu.VMEM((CH,), jnp.float32),
            pltpu.VMEM((2 * _L,), jnp.float32),
            pltpu.SemaphoreType.DMA,
        ],
    )


@jax.jit
def kernel(output, mask, ind, target):
    B, C, D, H, W = output.shape
    K = ind.shape[1]
    N = D * H * W
    WPB = _NW // B
    CH = -(-K // WPB)
    CH = -(CH // -128) * 128
    KPAD = CH * WPB

    flat = output.reshape(B * C * N)
    pad = ((0, 0), (0, KPAD - K))
    indf = jnp.pad(ind.astype(jnp.float32), pad).reshape(_NW, 1, CH)
    mk = jnp.pad(mask, pad).astype(jnp.float32).reshape(_NW, 1, CH)
    tg = jnp.pad(jnp.transpose(target, (0, 2, 1)),
                 ((0, 0), (0, 0), (0, KPAD - K)))
    tg = tg.reshape(B, C, WPB, CH).transpose(0, 2, 1, 3).reshape(_NW, C, CH)
    rest = jnp.concatenate([mk, tg], axis=1).reshape(-1)

    fn = _make_sc_kernel(B, C, N, KPAD, CH)
    parts = fn(flat, indf.reshape(-1), rest).reshape(_NW, 2, _L)
    lsum = jnp.sum(parts[:, 0, :])
    msum = jnp.sum(parts[:, 1, :])
    return lsum / (msum + 1e-4)


# speedup vs baseline: 1.1751x; 1.1751x over previous
import jax
import jax.numpy as jnp
from jax import lax
from jax.experimental import pallas as pl
from jax.experimental.pallas import tpu as pltpu
from jax.experimental.pallas import tpu_sc as plsc

_L = 16
_NW = 16


def _perm_gather(x, perm):
    dnums = lax.GatherDimensionNumbers(
        offset_dims=(), collapsed_slice_dims=(0,), start_index_map=(0,))
    return lax.gather(x, perm[:, None], dnums, slice_sizes=(1,),
                      mode=lax.GatherScatterMode.PROMISE_IN_BOUNDS)


def _lane_sum(x):
    lanes = lax.broadcasted_iota(jnp.int32, (_L,), 0)
    for k in (8, 4, 2, 1):
        x = x + _perm_gather(x, (lanes + k) & (_L - 1))
    return x


def _make_sc_kernel(B, C, N, KPAD, CH):
    WPB = KPAD // CH
    NV = CH // _L

    def body(flat_h, ind_h, rest_h, out_h,
             ind_v, rest_v, idx0, idx1, val0, val1, part_v, shared, accbuf,
             out_v, sem):
        w = lax.axis_index("s")
        b = w // WPB
        cin = pltpu.async_copy(ind_h.at[pl.ds(w * CH, CH)], ind_v, sem)
        crest = pltpu.async_copy(rest_h.at[pl.ds(w * 3 * CH, 3 * CH)],
                                 rest_v, sem)
        cin.wait()

        off0 = (b * C + 0) * N
        off1 = (b * C + 1) * N

        def build(j, _):
            n = ind_v[pl.ds(j * _L, _L)].astype(jnp.int32)
            idx0[pl.ds(j * _L, _L)] = n + off0
            idx1[pl.ds(j * _L, _L)] = n + off1
            return 0

        lax.fori_loop(0, NV, build, 0, unroll=False)

        cp0 = pltpu.async_copy(flat_h.at[idx0], val0, sem)
        cp1 = pltpu.async_copy(flat_h.at[idx1], val1, sem)
        crest.wait()
        cp0.wait()
        cp1.wait()

        def accum(j, carry):
            accl, accm = carry
            o = j * _L
            mk = rest_v[pl.ds(o, _L)]
            t0 = rest_v[pl.ds(CH + o, _L)]
            t1 = rest_v[pl.ds(2 * CH + o, _L)]
            accl = accl + (jnp.abs(val0[pl.ds(o, _L)] - t0)
                           + jnp.abs(val1[pl.ds(o, _L)] - t1)) * mk
            accm = accm + mk + mk
            return accl, accm

        zero = jnp.zeros((_L,), jnp.float32)
        accl, accm = lax.fori_loop(0, NV, accum, (zero, zero), unroll=False)

        part_v[pl.ds(0, _L)] = accl
        part_v[pl.ds(_L, _L)] = accm
        pltpu.sync_copy(part_v, shared.at[pl.ds(w * 2 * _L, 2 * _L)])
        plsc.subcore_barrier()

        @pl.when(w == 0)
        def _():
            pltpu.sync_copy(shared, accbuf)

            def comb(i, carry):
                al, am = carry
                return (al + accbuf[pl.ds(i * 2 * _L, _L)],
                        am + accbuf[pl.ds(i * 2 * _L + _L, _L)])

            al, am = lax.fori_loop(0, _NW, comb, (zero, zero), unroll=False)
            al = _lane_sum(al)
            am = _lane_sum(am)
            out_v[...] = al / (am + 1e-4)
            pltpu.sync_copy(out_v, out_h)

    mesh = plsc.VectorSubcoreMesh(
        core_axis_name="c", subcore_axis_name="s", num_cores=1,
        num_subcores=_NW)
    return pl.kernel(
        body,
        out_type=jax.ShapeDtypeStruct((_L,), jnp.float32),
        mesh=mesh,
        scratch_types=[
            pltpu.VMEM((CH,), jnp.float32),
            pltpu.VMEM((3 * CH,), jnp.float32),
            pltpu.VMEM((CH,), jnp.int32),
            pltpu.VMEM((CH,), jnp.int32),
            pltpu.VMEM((CH,), jnp.float32),
            pltpu.VMEM((CH,), jnp.float32),
            pltpu.VMEM((2 * _L,), jnp.float32),
            pltpu.VMEM_SHARED((_NW * 2 * _L,), jnp.float32),
            pltpu.VMEM((_NW * 2 * _L,), jnp.float32),
            pltpu.VMEM((_L,), jnp.float32),
            pltpu.SemaphoreType.DMA,
        ],
    )


@jax.jit
def kernel(output, mask, ind, target):
    B, C, D, H, W = output.shape
    K = ind.shape[1]
    N = D * H * W
    WPB = _NW // B
    CH = -(-K // WPB)
    CH = -(CH // -128) * 128
    KPAD = CH * WPB

    flat = output.reshape(B * C * N)
    pad = ((0, 0), (0, KPAD - K))
    indf = jnp.pad(ind.astype(jnp.float32), pad).reshape(_NW, 1, CH)
    mk = jnp.pad(mask, pad).astype(jnp.float32).reshape(_NW, 1, CH)
    tg = jnp.pad(jnp.transpose(target, (0, 2, 1)),
                 ((0, 0), (0, 0), (0, KPAD - K)))
    tg = tg.reshape(B, C, WPB, CH).transpose(0, 2, 1, 3).reshape(_NW, C, CH)
    rest = jnp.concatenate([mk, tg], axis=1).reshape(-1)

    fn = _make_sc_kernel(B, C, N, KPAD, CH)
    res = fn(flat, indf.reshape(-1), rest)
    return res[0]
